# baseline (device time: 24976 ns/iter reference)
import jax
import jax.numpy as jnp
from jax import lax
from jax.experimental import pallas as pl
from jax.experimental.pallas import tpu as pltpu

N_DEV = 4
B = 2
SQ_LOC = 128
SKV = 512
HQ, DH = 4, 64
HD = HQ * DH
E = 512
WINDOW = 128
NGLOB = 32


def kernel(x, Wq, K_ext, V_ext, Wo):
    K2 = K_ext.reshape(B, SQ_LOC, HD)
    V2 = V_ext.reshape(B, SQ_LOC, HD)

    def body(x_ref, wq_ref, k_ref, v_ref, wo_ref, out_ref,
             kv_all, send_sems, recv_sems):
        p = lax.axis_index("i")
        left = lax.rem(p + N_DEV - 1, N_DEV)
        right = lax.rem(p + 1, N_DEV)

        barrier_sem = pltpu.get_barrier_semaphore()
        for nbr in (left, right):
            pl.semaphore_signal(
                barrier_sem, inc=1,
                device_id=(nbr,), device_id_type=pl.DeviceIdType.MESH,
            )
        pl.semaphore_wait(barrier_sem, 2)

        my_rows = pl.ds(p * SQ_LOC, SQ_LOC)
        kv_all[:, my_rows, 0:HD] = k_ref[...].astype(jnp.bfloat16)
        kv_all[:, my_rows, HD:2 * HD] = v_ref[...].astype(jnp.bfloat16)

        for h in range(N_DEV - 1):
            origin = lax.rem(p + N_DEV - h, N_DEV)
            rows = pl.ds(origin * SQ_LOC, SQ_LOC)
            rdma = pltpu.make_async_remote_copy(
                src_ref=kv_all.at[:, rows, :],
                dst_ref=kv_all.at[:, rows, :],
                send_sem=send_sems.at[h],
                recv_sem=recv_sems.at[h],
                device_id=(right,),
                device_id_type=pl.DeviceIdType.MESH,
            )
            rdma.start()
            rdma.wait()

        qi = lax.broadcasted_iota(jnp.int32, (SQ_LOC, SKV), 0) + p * SQ_LOC
        ki = lax.broadcasted_iota(jnp.int32, (SQ_LOC, SKV), 1)
        mask = (jnp.abs(qi - ki) <= WINDOW) | (ki < NGLOB) | (qi < NGLOB)

        wq = wq_ref[...].astype(jnp.bfloat16)
        wo = wo_ref[...].astype(jnp.bfloat16)
        for b in range(B):
            xb = x_ref[b].astype(jnp.bfloat16)
            qb = jnp.dot(xb, wq, preferred_element_type=jnp.float32)
            ctx_heads = []
            for h in range(HQ):
                q = qb[:, h * DH:(h + 1) * DH].astype(jnp.bfloat16)
                k = kv_all[b, :, h * DH:(h + 1) * DH]
                v = kv_all[b, :, HD + h * DH:HD + (h + 1) * DH]
                s = lax.dot_general(
                    q, k, (((1,), (1,)), ((), ())),
                    preferred_element_type=jnp.float32,
                ) * 0.125
                s = jnp.where(mask, s, -1e9)
                m = jnp.max(s, axis=1, keepdims=True)
                w = jnp.exp(s - m)
                w = w / jnp.sum(w, axis=1, keepdims=True)
                ctx_heads.append(
                    jnp.dot(w.astype(jnp.bfloat16), v,
                            preferred_element_type=jnp.float32)
                )
            ctx = jnp.concatenate(ctx_heads, axis=1).astype(jnp.bfloat16)
            out_ref[b] = jnp.dot(ctx, wo, preferred_element_type=jnp.float32)

    return pl.pallas_call(
        body,
        out_shape=jax.ShapeDtypeStruct((B, SQ_LOC, E), jnp.float32),
        in_specs=[pl.BlockSpec(memory_space=pltpu.VMEM)] * 5,
        out_specs=pl.BlockSpec(memory_space=pltpu.VMEM),
        scratch_shapes=[
            pltpu.VMEM((B, SKV, 2 * HD), jnp.bfloat16),
            pltpu.SemaphoreType.DMA((N_DEV - 1,)),
            pltpu.SemaphoreType.DMA((N_DEV - 1,)),
        ],
        compiler_params=pltpu.CompilerParams(collective_id=0),
    )(x, Wq, K2, V2, Wo)


# device time: 18576 ns/iter; 1.3445x vs baseline; 1.3445x over previous
import jax
import jax.numpy as jnp
from jax import lax
from jax.experimental import pallas as pl
from jax.experimental.pallas import tpu as pltpu

N_DEV = 4
B = 2
SQ_LOC = 128
SKV = 512
HQ, DH = 4, 64
HD = HQ * DH
E = 512
WINDOW = 128
NGLOB = 32


def kernel(x, Wq, K_ext, V_ext, Wo):
    K2 = K_ext.reshape(B, SQ_LOC, HD)
    V2 = V_ext.reshape(B, SQ_LOC, HD)

    def body(x_ref, wq_ref, k_ref, v_ref, wo_ref, out_ref,
             kv_all, send_sems, recv_sems):
        p = lax.axis_index("i")
        left = lax.rem(p + N_DEV - 1, N_DEV)
        right = lax.rem(p + 1, N_DEV)
        opp = lax.rem(p + 2, N_DEV)

        barrier_sem = pltpu.get_barrier_semaphore()
        for nbr in (left, right, opp):
            pl.semaphore_signal(
                barrier_sem, inc=1,
                device_id=(nbr,), device_id_type=pl.DeviceIdType.MESH,
            )
        pl.semaphore_wait(barrier_sem, 3)

        my_rows = pl.ds(p * SQ_LOC, SQ_LOC)
        kv_all[:, my_rows, 0:HD] = k_ref[...].astype(jnp.bfloat16)
        kv_all[:, my_rows, HD:2 * HD] = v_ref[...].astype(jnp.bfloat16)

        sends = []
        for tgt, idx in ((right, 0), (left, 1), (opp, 2)):
            rdma = pltpu.make_async_remote_copy(
                src_ref=kv_all.at[:, my_rows, :],
                dst_ref=kv_all.at[:, my_rows, :],
                send_sem=send_sems.at[idx],
                recv_sem=recv_sems.at[idx],
                device_id=(tgt,),
                device_id_type=pl.DeviceIdType.MESH,
            )
            rdma.start()
            sends.append(rdma)

        qi = lax.broadcasted_iota(jnp.int32, (SQ_LOC, SKV), 0) + p * SQ_LOC
        ki = lax.broadcasted_iota(jnp.int32, (SQ_LOC, SKV), 1)
        mask = (jnp.abs(qi - ki) <= WINDOW) | (ki < NGLOB) | (qi < NGLOB)

        wq = wq_ref[...].astype(jnp.bfloat16)
        wo = wo_ref[...].astype(jnp.bfloat16)
        qproj = [
            jnp.dot(x_ref[b].astype(jnp.bfloat16), wq,
                    preferred_element_type=jnp.float32)
            for b in range(B)
        ]

        for src_pos, idx in ((left, 0), (right, 1), (opp, 2)):
            rows = pl.ds(src_pos * SQ_LOC, SQ_LOC)
            pltpu.make_async_remote_copy(
                src_ref=kv_all.at[:, rows, :],
                dst_ref=kv_all.at[:, rows, :],
                send_sem=send_sems.at[idx],
                recv_sem=recv_sems.at[idx],
                device_id=(src_pos,),
                device_id_type=pl.DeviceIdType.MESH,
            ).wait_recv()

        for b in range(B):
            qb = qproj[b]
            ctx_heads = []
            for h in range(HQ):
                q = qb[:, h * DH:(h + 1) * DH].astype(jnp.bfloat16)
                k = kv_all[b, :, h * DH:(h + 1) * DH]
                v = kv_all[b, :, HD + h * DH:HD + (h + 1) * DH]
                s = lax.dot_general(
                    q, k, (((1,), (1,)), ((), ())),
                    preferred_element_type=jnp.float32,
                ) * 0.125
                s = jnp.where(mask, s, -1e9)
                m = jnp.max(s, axis=1, keepdims=True)
                w = jnp.exp(s - m)
                w = w / jnp.sum(w, axis=1, keepdims=True)
                ctx_heads.append(
                    jnp.dot(w.astype(jnp.bfloat16), v,
                            preferred_element_type=jnp.float32)
                )
            ctx = jnp.concatenate(ctx_heads, axis=1).astype(jnp.bfloat16)
            out_ref[b] = jnp.dot(ctx, wo, preferred_element_type=jnp.float32)

        for rdma in sends:
            rdma.wait_send()

    return pl.pallas_call(
        body,
        out_shape=jax.ShapeDtypeStruct((B, SQ_LOC, E), jnp.float32),
        in_specs=[pl.BlockSpec(memory_space=pltpu.VMEM)] * 5,
        out_specs=pl.BlockSpec(memory_space=pltpu.VMEM),
        scratch_shapes=[
            pltpu.VMEM((B, SKV, 2 * HD), jnp.bfloat16),
            pltpu.SemaphoreType.DMA((3,)),
            pltpu.SemaphoreType.DMA((3,)),
        ],
        compiler_params=pltpu.CompilerParams(collective_id=0),
    )(x, Wq, K2, V2, Wo)


# device time: 17054 ns/iter; 1.4645x vs baseline; 1.0892x over previous
import jax
import jax.numpy as jnp
from jax import lax
from jax.experimental import pallas as pl
from jax.experimental.pallas import tpu as pltpu

N_DEV = 4
B = 2
SQ_LOC = 128
SKV = 512
HQ, DH = 4, 64
HD = HQ * DH
E = 512
WINDOW = 128
NGLOB = 32


def kernel(x, Wq, K_ext, V_ext, Wo):
    K2 = K_ext.reshape(B, SQ_LOC, HD)
    V2 = V_ext.reshape(B, SQ_LOC, HD)

    def body(x_ref, wq_ref, k_ref, v_ref, wo_ref, out_ref,
             kv_all, send_sems, recv_sems):
        p = lax.axis_index("i")
        left = lax.rem(p + N_DEV - 1, N_DEV)
        right = lax.rem(p + 1, N_DEV)
        opp = lax.rem(p + 2, N_DEV)

        barrier_sem = pltpu.get_barrier_semaphore()
        for nbr in (left, right, opp):
            pl.semaphore_signal(
                barrier_sem, inc=1,
                device_id=(nbr,), device_id_type=pl.DeviceIdType.MESH,
            )
        pl.semaphore_wait(barrier_sem, 3)

        my_rows = pl.ds(p * SQ_LOC, SQ_LOC)
        kv_all[:, my_rows, 0:HD] = k_ref[...].astype(jnp.bfloat16)
        kv_all[:, my_rows, HD:2 * HD] = v_ref[...].astype(jnp.bfloat16)

        sends = []
        for tgt, idx in ((right, 0), (left, 1), (opp, 2)):
            rdma = pltpu.make_async_remote_copy(
                src_ref=kv_all.at[:, my_rows, :],
                dst_ref=kv_all.at[:, my_rows, :],
                send_sem=send_sems.at[idx],
                recv_sem=recv_sems.at[idx],
                device_id=(tgt,),
                device_id_type=pl.DeviceIdType.MESH,
            )
            rdma.start()
            sends.append(rdma)

        qi = lax.broadcasted_iota(jnp.int32, (SQ_LOC, SKV), 0) + p * SQ_LOC
        ki = lax.broadcasted_iota(jnp.int32, (SQ_LOC, SKV), 1)
        mask = (jnp.abs(qi - ki) <= WINDOW) | (ki < NGLOB) | (qi < NGLOB)
        bias = jnp.where(mask, 0.0, -1e9).astype(jnp.float32)

        wq = wq_ref[...].astype(jnp.bfloat16)
        wo = wo_ref[...].astype(jnp.bfloat16)
        qproj = [
            (jnp.dot(x_ref[b].astype(jnp.bfloat16), wq,
                     preferred_element_type=jnp.float32) * 0.125)
            for b in range(B)
        ]

        for src_pos, idx in ((left, 0), (right, 1), (opp, 2)):
            rows = pl.ds(src_pos * SQ_LOC, SQ_LOC)
            pltpu.make_async_remote_copy(
                src_ref=kv_all.at[:, rows, :],
                dst_ref=kv_all.at[:, rows, :],
                send_sem=send_sems.at[idx],
                recv_sem=recv_sems.at[idx],
                device_id=(src_pos,),
                device_id_type=pl.DeviceIdType.MESH,
            ).wait_recv()

        for b in range(B):
            qb = qproj[b]
            ctx_heads = []
            for h in range(HQ):
                q = qb[:, h * DH:(h + 1) * DH].astype(jnp.bfloat16)
                k = kv_all[b, :, h * DH:(h + 1) * DH]
                v = kv_all[b, :, HD + h * DH:HD + (h + 1) * DH]
                s = lax.dot_general(
                    q, k, (((1,), (1,)), ((), ())),
                    preferred_element_type=jnp.float32,
                )
                w = jnp.exp(s + bias)
                recip = 1.0 / jnp.sum(w, axis=1, keepdims=True)
                ctx_h = jnp.dot(w.astype(jnp.bfloat16), v,
                                preferred_element_type=jnp.float32)
                ctx_heads.append(ctx_h * recip)
            ctx = jnp.concatenate(ctx_heads, axis=1).astype(jnp.bfloat16)
            out_ref[b] = jnp.dot(ctx, wo, preferred_element_type=jnp.float32)

        for rdma in sends:
            rdma.wait_send()

    return pl.pallas_call(
        body,
        out_shape=jax.ShapeDtypeStruct((B, SQ_LOC, E), jnp.float32),
        in_specs=[pl.BlockSpec(memory_space=pltpu.VMEM)] * 5,
        out_specs=pl.BlockSpec(memory_space=pltpu.VMEM),
        scratch_shapes=[
            pltpu.VMEM((B, SKV, 2 * HD), jnp.bfloat16),
            pltpu.SemaphoreType.DMA((3,)),
            pltpu.SemaphoreType.DMA((3,)),
        ],
        compiler_params=pltpu.CompilerParams(collective_id=0),
    )(x, Wq, K2, V2, Wo)
